# baseline (speedup 1.0000x reference)
import jax
import jax.numpy as jnp
from jax.experimental import pallas as pl
from jax.experimental.pallas import tpu as pltpu
from jax.experimental.pallas import tpu_sc as plsc

_N = 1024
_D = 64
_T = 12
_B = 16
_K = 5

_NT = 16
_NC = 2
_SLAB = _N * _N
_SLICE = _SLAB // _NT
_UPT = _N * 2 * _K // _NT
_G = _UPT // 128


def _topk_body(x_ref, idx_ref, val_ref):
    xcat = jnp.concatenate([x_ref[t, 0] for t in range(_T)], axis=1)
    nsq = jnp.sum(xcat * xcat, axis=1, keepdims=True)
    xn = xcat * jax.lax.rsqrt(nsq)
    acc = jax.lax.dot_general(
        xn, xn, (((1,), (1,)), ((), ())),
        preferred_element_type=jnp.float32)

    col_i = jax.lax.broadcasted_iota(jnp.int32, (_N, _N), 1)
    row_i = jax.lax.broadcasted_iota(jnp.int32, (_N, 1), 0)
    work = acc
    jcols, vcols = [], []
    for _ in range(_K):
        m = jnp.max(work, axis=1, keepdims=True)
        ji = jnp.min(jnp.where(work >= m, col_i, 2 * _N),
                     axis=1, keepdims=True)
        lv = jnp.where(m >= 0, m, 0.01 * m) * 0.5
        jcols.append(ji)
        vcols.append(lv)
        work = jnp.where(col_i == ji, -jnp.inf, work)
    is_self = jcols[0] == row_i
    dvals = [jnp.where(is_self, vcols[0] + vcols[0], vcols[0])] + vcols[1:]
    tvals = [jnp.where(is_self, 0.0, vcols[0])] + vcols[1:]
    icols = [row_i * _N + ji for ji in jcols]
    tcols = [ji * _N + row_i for ji in jcols]
    zi = jnp.zeros((_N, 1), jnp.int32)
    zf = jnp.zeros((_N, 1), jnp.float32)
    u_idx = jnp.concatenate(icols + tcols + [zi] * 6, axis=1)
    u_val = jnp.concatenate(dvals + tvals + [zf] * 6, axis=1)
    idx_ref[0] = u_idx.T
    val_ref[0] = u_val.T


_NPLANES = 16


def _sc_scatter(idx_hbm, val_hbm, out_hbm, idx_v, val_v, chunk):
    c = jax.lax.axis_index("c")
    s = jax.lax.axis_index("s")
    base = s * _SLICE

    def zbody(i, carry):
        chunk[pl.ds(i * 16, 16)] = jnp.zeros((16,), jnp.float32)
        return carry
    jax.lax.fori_loop(0, _SLICE // 16, zbody, 0, unroll=8)

    for r in range(_B // _NC):
        b = r * _NC + c
        pltpu.sync_copy(idx_hbm.at[b], idx_v)
        pltpu.sync_copy(val_hbm.at[b], val_v)

        for p in range(_NPLANES):
            def sbody(i, carry):
                iv = idx_v[p, pl.ds(i * 16, 16)] - base
                vv = val_v[p, pl.ds(i * 16, 16)]
                msk = (iv >= 0) & (iv < _SLICE)
                loc = jnp.where(msk, iv, 0)
                plsc.addupdate_scatter(chunk, [loc], jnp.where(msk, vv, 0.0))
                return carry
            jax.lax.fori_loop(0, _N // 16, sbody, 0, unroll=4)

        pltpu.sync_copy(chunk, out_hbm.at[b, pl.ds(base, _SLICE)])

        for p in range(_NPLANES):
            def wbody(i, carry):
                iv = idx_v[p, pl.ds(i * 16, 16)] - base
                msk = (iv >= 0) & (iv < _SLICE)
                loc = jnp.where(msk, iv, 0)
                plsc.store_scatter(chunk, [loc],
                                   jnp.zeros((16,), jnp.float32), mask=msk)
                return carry
            jax.lax.fori_loop(0, _N // 16, wbody, 0, unroll=4)


_scatter_call = pl.kernel(
    _sc_scatter,
    out_type=jax.ShapeDtypeStruct((_B, _SLAB), jnp.float32),
    mesh=plsc.VectorSubcoreMesh(core_axis_name="c", subcore_axis_name="s"),
    compiler_params=pltpu.CompilerParams(needs_layout_passes=False),
    scratch_types=[
        pltpu.VMEM((_NPLANES, _N), jnp.int32),
        pltpu.VMEM((_NPLANES, _N), jnp.float32),
        pltpu.VMEM((_SLICE,), jnp.float32),
    ],
)


def kernel(x):
    idx, val = pl.pallas_call(
        _topk_body,
        grid=(_B,),
        in_specs=[pl.BlockSpec((_T, 1, _N, _D), lambda b: (0, b, 0, 0))],
        out_specs=[
            pl.BlockSpec((1, _NPLANES, _N), lambda b: (b, 0, 0)),
            pl.BlockSpec((1, _NPLANES, _N), lambda b: (b, 0, 0)),
        ],
        out_shape=[
            jax.ShapeDtypeStruct((_B, _NPLANES, _N), jnp.int32),
            jax.ShapeDtypeStruct((_B, _NPLANES, _N), jnp.float32),
        ],
    )(x)
    flat = _scatter_call(idx, val)
    return flat.reshape(_B, _N, _N)


# speedup vs baseline: 1.2511x; 1.2511x over previous
import jax
import jax.numpy as jnp
from jax.experimental import pallas as pl
from jax.experimental.pallas import tpu as pltpu
from jax.experimental.pallas import tpu_sc as plsc

_N = 1024
_D = 64
_T = 12
_B = 16
_K = 5

_NT = 16
_NC = 2
_SLAB = _N * _N
_SLICE = _SLAB // _NT
_UPT = _N * 2 * _K // _NT
_G = _UPT // 128


def _topk_body(x_ref, idx_ref, val_ref):
    xcat = jnp.concatenate([x_ref[t, 0] for t in range(_T)], axis=1)
    nsq = jnp.sum(xcat * xcat, axis=1, keepdims=True)
    xn = xcat * jax.lax.rsqrt(nsq)
    acc = jax.lax.dot_general(
        xn, xn, (((1,), (1,)), ((), ())),
        preferred_element_type=jnp.float32)

    colf = jax.lax.broadcasted_iota(
        jnp.int32, (_N, _N), 1).astype(jnp.float32)
    rowf = jax.lax.broadcasted_iota(
        jnp.int32, (_N, 1), 0).astype(jnp.float32)
    work = acc
    jcols, vcols = [], []
    for _ in range(_K):
        m = jnp.max(work, axis=1, keepdims=True)
        eq = work >= m
        jf = jnp.max(jnp.where(eq, colf, -1.0), axis=1, keepdims=True)
        lv = jnp.where(m >= 0, m, 0.01 * m) * 0.5
        jcols.append(jf)
        vcols.append(lv)
        work = jnp.where(eq, -jnp.inf, work)
    is_self = jcols[0] == rowf
    dvals = [jnp.where(is_self, vcols[0] + vcols[0], vcols[0])] + vcols[1:]
    tvals = [jnp.where(is_self, 0.0, vcols[0])] + vcols[1:]
    fn = float(_N)
    icols = [rowf * fn + jf for jf in jcols]
    tcols = [jf * fn + rowf for jf in jcols]
    zf = jnp.zeros((_N, 1), jnp.float32)
    u_idx = jnp.concatenate(icols + tcols + [zf] * 6, axis=1)
    u_val = jnp.concatenate(dvals + tvals + [zf] * 6, axis=1)
    idx_ref[0] = u_idx.T.astype(jnp.int32)
    val_ref[0] = u_val.T


_NPLANES = 16


def _sc_scatter(idx_hbm, val_hbm, out_hbm, idx_v, val_v, chunk):
    c = jax.lax.axis_index("c")
    s = jax.lax.axis_index("s")
    base = s * _SLICE

    def zbody(i, carry):
        chunk[pl.ds(i * 16, 16)] = jnp.zeros((16,), jnp.float32)
        return carry
    jax.lax.fori_loop(0, _SLICE // 16, zbody, 0, unroll=8)

    for r in range(_B // _NC):
        b = r * _NC + c
        pltpu.sync_copy(idx_hbm.at[b], idx_v)
        pltpu.sync_copy(val_hbm.at[b], val_v)

        for p in range(2 * _K):
            def sbody(i, carry):
                iv = idx_v[p, pl.ds(i * 16, 16)] - base
                vv = val_v[p, pl.ds(i * 16, 16)]
                msk = (iv >= 0) & (iv < _SLICE)
                loc = jnp.where(msk, iv, 0)
                plsc.addupdate_scatter(chunk, [loc], jnp.where(msk, vv, 0.0))
                return carry
            jax.lax.fori_loop(0, _N // 16, sbody, 0, unroll=4)

        pltpu.sync_copy(chunk, out_hbm.at[b, pl.ds(base, _SLICE)])

        for p in range(2 * _K):
            def wbody(i, carry):
                iv = idx_v[p, pl.ds(i * 16, 16)] - base
                msk = (iv >= 0) & (iv < _SLICE)
                loc = jnp.where(msk, iv, 0)
                plsc.store_scatter(chunk, [loc],
                                   jnp.zeros((16,), jnp.float32), mask=msk)
                return carry
            jax.lax.fori_loop(0, _N // 16, wbody, 0, unroll=4)


_scatter_call = pl.kernel(
    _sc_scatter,
    out_type=jax.ShapeDtypeStruct((_B, _SLAB), jnp.float32),
    mesh=plsc.VectorSubcoreMesh(core_axis_name="c", subcore_axis_name="s"),
    compiler_params=pltpu.CompilerParams(needs_layout_passes=False),
    scratch_types=[
        pltpu.VMEM((_NPLANES, _N), jnp.int32),
        pltpu.VMEM((_NPLANES, _N), jnp.float32),
        pltpu.VMEM((_SLICE,), jnp.float32),
    ],
)


def kernel(x):
    idx, val = pl.pallas_call(
        _topk_body,
        grid=(_B,),
        in_specs=[pl.BlockSpec((_T, 1, _N, _D), lambda b: (0, b, 0, 0))],
        out_specs=[
            pl.BlockSpec((1, _NPLANES, _N), lambda b: (b, 0, 0)),
            pl.BlockSpec((1, _NPLANES, _N), lambda b: (b, 0, 0)),
        ],
        out_shape=[
            jax.ShapeDtypeStruct((_B, _NPLANES, _N), jnp.int32),
            jax.ShapeDtypeStruct((_B, _NPLANES, _N), jnp.float32),
        ],
    )(x)
    flat = _scatter_call(idx, val)
    return flat.reshape(_B, _N, _N)


# speedup vs baseline: 1.5001x; 1.1990x over previous
import jax
import jax.numpy as jnp
from jax.experimental import pallas as pl
from jax.experimental.pallas import tpu as pltpu
from jax.experimental.pallas import tpu_sc as plsc

_N = 1024
_D = 64
_T = 12
_B = 16
_K = 5

_NT = 16
_NC = 2
_SLAB = _N * _N
_SLICE = _SLAB // _NT
_UPT = _N * 2 * _K // _NT
_G = _UPT // 128


def _topk_body(x_ref, idx_ref, val_ref):
    xcat = jnp.concatenate([x_ref[t, 0] for t in range(_T)], axis=1)
    nsq = jnp.sum(xcat * xcat, axis=1, keepdims=True)
    xn = xcat * jax.lax.rsqrt(nsq)
    acc = jax.lax.dot_general(
        xn, xn, (((1,), (1,)), ((), ())),
        preferred_element_type=jnp.float32)

    colf = jax.lax.broadcasted_iota(
        jnp.int32, (_N, _N), 1).astype(jnp.float32)
    rowf = jax.lax.broadcasted_iota(
        jnp.int32, (_N, 1), 0).astype(jnp.float32)
    work = acc
    jcols, vcols = [], []
    for _ in range(_K):
        m = jnp.max(work, axis=1, keepdims=True)
        eq = work >= m
        jf = jnp.max(jnp.where(eq, colf, -1.0), axis=1, keepdims=True)
        lv = jnp.where(m >= 0, m, 0.01 * m) * 0.5
        jcols.append(jf)
        vcols.append(lv)
        work = jnp.where(eq, -jnp.inf, work)
    is_self = jcols[0] == rowf
    dvals = [jnp.where(is_self, vcols[0] + vcols[0], vcols[0])] + vcols[1:]
    tvals = [jnp.where(is_self, 0.0, vcols[0])] + vcols[1:]
    fn = float(_N)
    icols = [rowf * fn + jf for jf in jcols]
    tcols = [jf * fn + rowf for jf in jcols]
    zf = jnp.zeros((_N, 1), jnp.float32)
    u_idx = jnp.concatenate(icols + tcols + [zf] * 6, axis=1)
    u_val = jnp.concatenate(dvals + tvals + [zf] * 6, axis=1)
    idx_ref[0] = u_idx.T.astype(jnp.int32)
    val_ref[0] = u_val.T


_NPLANES = 16


_RPT = _N // _NT


def _sc_scatter(idx_hbm, val_hbm, out_hbm, idx_v, val_v, chunk):
    c = jax.lax.axis_index("c")
    s = jax.lax.axis_index("s")
    base = s * _SLICE

    def zbody(i, carry):
        chunk[pl.ds(i * 16, 16)] = jnp.zeros((16,), jnp.float32)
        return carry

    jax.lax.fori_loop(0, _SLICE // 16, zbody, 0, unroll=8)

    for r in range(_B // _NC):
        b = r * _NC + c
        pltpu.sync_copy(idx_hbm.at[b], idx_v)
        pltpu.sync_copy(val_hbm.at[b], val_v)

        for p in range(_K):
            def dbody(i, carry):
                j = (i + s * (_RPT // 16)) * 16
                iv = idx_v[p, pl.ds(j, 16)] - base
                vv = val_v[p, pl.ds(j, 16)]
                plsc.addupdate_scatter(chunk, [iv], vv)
                return carry
            jax.lax.fori_loop(0, _RPT // 16, dbody, 0, unroll=4)

            def tbody(i, carry):
                iv = idx_v[_K + p, pl.ds(i * 16, 16)] - base
                vv = val_v[_K + p, pl.ds(i * 16, 16)]
                msk = (iv >= 0) & (iv < _SLICE)
                loc = jnp.where(msk, iv, 0)
                plsc.addupdate_scatter(chunk, [loc], jnp.where(msk, vv, 0.0))
                return carry
            jax.lax.fori_loop(0, _N // 16, tbody, 0, unroll=8)

        pltpu.sync_copy(chunk, out_hbm.at[b, pl.ds(base, _SLICE)])
        jax.lax.fori_loop(0, _SLICE // 16, zbody, 0, unroll=8)


_scatter_call = pl.kernel(
    _sc_scatter,
    out_type=jax.ShapeDtypeStruct((_B, _SLAB), jnp.float32),
    mesh=plsc.VectorSubcoreMesh(core_axis_name="c", subcore_axis_name="s"),
    compiler_params=pltpu.CompilerParams(needs_layout_passes=False),
    scratch_types=[
        pltpu.VMEM((_NPLANES, _N), jnp.int32),
        pltpu.VMEM((_NPLANES, _N), jnp.float32),
        pltpu.VMEM((_SLICE,), jnp.float32),
    ],
)


def kernel(x):
    idx, val = pl.pallas_call(
        _topk_body,
        grid=(_B,),
        in_specs=[pl.BlockSpec((_T, 1, _N, _D), lambda b: (0, b, 0, 0))],
        out_specs=[
            pl.BlockSpec((1, _NPLANES, _N), lambda b: (b, 0, 0)),
            pl.BlockSpec((1, _NPLANES, _N), lambda b: (b, 0, 0)),
        ],
        out_shape=[
            jax.ShapeDtypeStruct((_B, _NPLANES, _N), jnp.int32),
            jax.ShapeDtypeStruct((_B, _NPLANES, _N), jnp.float32),
        ],
    )(x)
    flat = _scatter_call(idx, val)
    return flat.reshape(_B, _N, _N)


# speedup vs baseline: 1.5522x; 1.0347x over previous
import jax
import jax.numpy as jnp
from jax.experimental import pallas as pl
from jax.experimental.pallas import tpu as pltpu
from jax.experimental.pallas import tpu_sc as plsc

_N = 1024
_D = 64
_T = 12
_B = 16
_K = 5

_NT = 16
_NC = 2
_SLAB = _N * _N
_SLICE = _SLAB // _NT
_UPT = _N * 2 * _K // _NT
_G = _UPT // 128


def _topk_body(x_ref, idx_ref, val_ref):
    xcat = jnp.concatenate([x_ref[t, 0] for t in range(_T)], axis=1)
    nsq = jnp.sum(xcat * xcat, axis=1, keepdims=True)
    xn = xcat * jax.lax.rsqrt(nsq)
    acc = jax.lax.dot_general(
        xn, xn, (((1,), (1,)), ((), ())),
        preferred_element_type=jnp.float32)

    colf = jax.lax.broadcasted_iota(
        jnp.int32, (_N, _N), 1).astype(jnp.float32)
    rowf = jax.lax.broadcasted_iota(
        jnp.int32, (_N, 1), 0).astype(jnp.float32)
    work = acc
    jcols, vcols = [], []
    for _ in range(_K):
        m = jnp.max(work, axis=1, keepdims=True)
        eq = work >= m
        jf = jnp.max(jnp.where(eq, colf, -1.0), axis=1, keepdims=True)
        lv = jnp.where(m >= 0, m, 0.01 * m) * 0.5
        jcols.append(jf)
        vcols.append(lv)
        work = jnp.where(eq, -jnp.inf, work)
    is_self = jcols[0] == rowf
    dvals = [jnp.where(is_self, vcols[0] + vcols[0], vcols[0])] + vcols[1:]
    tvals = [jnp.where(is_self, 0.0, vcols[0])] + vcols[1:]
    fn = float(_N)
    icols = [rowf * fn + jf for jf in jcols]
    tcols = [jf * fn + rowf for jf in jcols]
    zf = jnp.zeros((_N, 1), jnp.float32)
    u_idx = jnp.concatenate(icols + tcols + [zf] * 6, axis=1)
    u_val = jnp.concatenate(dvals + tvals + [zf] * 6, axis=1)
    idx_ref[0] = u_idx.T.astype(jnp.int32)
    val_ref[0] = u_val.T


_NPLANES = 16


_RPT = _N // _NT
_NG = 2
_GB = _B // _NG


def _sc_scatter(idx_hbm, val_hbm, out_hbm, idx_v, val_v, chunk):
    c = jax.lax.axis_index("c")
    s = jax.lax.axis_index("s")
    base = s * _SLICE

    def zbody(i, carry):
        chunk[pl.ds(i * 16, 16)] = jnp.zeros((16,), jnp.float32)
        return carry

    jax.lax.fori_loop(0, _SLICE // 16, zbody, 0, unroll=8)

    for r in range(_GB // _NC):
        b = r * _NC + c
        pltpu.sync_copy(idx_hbm.at[b], idx_v)
        pltpu.sync_copy(val_hbm.at[b], val_v)

        for p in range(_K):
            def dbody(i, carry):
                j = (i + s * (_RPT // 16)) * 16
                iv = idx_v[p, pl.ds(j, 16)] - base
                vv = val_v[p, pl.ds(j, 16)]
                plsc.addupdate_scatter(chunk, [iv], vv)
                return carry
            jax.lax.fori_loop(0, _RPT // 16, dbody, 0, unroll=4)

            def tbody(i, carry):
                iv = idx_v[_K + p, pl.ds(i * 16, 16)] - base
                vv = val_v[_K + p, pl.ds(i * 16, 16)]
                msk = (iv >= 0) & (iv < _SLICE)
                loc = jnp.where(msk, iv, 0)
                plsc.addupdate_scatter(chunk, [loc], jnp.where(msk, vv, 0.0))
                return carry
            jax.lax.fori_loop(0, _N // 16, tbody, 0, unroll=8)

        pltpu.sync_copy(chunk, out_hbm.at[b, pl.ds(base, _SLICE)])
        jax.lax.fori_loop(0, _SLICE // 16, zbody, 0, unroll=8)


_scatter_call = pl.kernel(
    _sc_scatter,
    out_type=jax.ShapeDtypeStruct((_GB, _SLAB), jnp.float32),
    mesh=plsc.VectorSubcoreMesh(core_axis_name="c", subcore_axis_name="s"),
    compiler_params=pltpu.CompilerParams(needs_layout_passes=False),
    scratch_types=[
        pltpu.VMEM((_NPLANES, _N), jnp.int32),
        pltpu.VMEM((_NPLANES, _N), jnp.float32),
        pltpu.VMEM((_SLICE,), jnp.float32),
    ],
)


def _topk_call(x, g):
    return pl.pallas_call(
        _topk_body,
        grid=(_GB,),
        in_specs=[pl.BlockSpec((_T, 1, _N, _D),
                               lambda b, g=g: (0, g * _GB + b, 0, 0))],
        out_specs=[
            pl.BlockSpec((1, _NPLANES, _N), lambda b: (b, 0, 0)),
            pl.BlockSpec((1, _NPLANES, _N), lambda b: (b, 0, 0)),
        ],
        out_shape=[
            jax.ShapeDtypeStruct((_GB, _NPLANES, _N), jnp.int32),
            jax.ShapeDtypeStruct((_GB, _NPLANES, _N), jnp.float32),
        ],
    )(x)


def kernel(x):
    flats = []
    for g in range(_NG):
        idx, val = _topk_call(x, g)
        flats.append(_scatter_call(idx, val))
    return jnp.concatenate(flats, axis=0).reshape(_B, _N, _N)


# speedup vs baseline: 1.6099x; 1.0372x over previous
import jax
import jax.numpy as jnp
from jax.experimental import pallas as pl
from jax.experimental.pallas import tpu as pltpu
from jax.experimental.pallas import tpu_sc as plsc

_N = 1024
_D = 64
_T = 12
_B = 16
_K = 5

_NT = 16
_NC = 2
_SLAB = _N * _N
_SLICE = _SLAB // _NT
_UPT = _N * 2 * _K // _NT
_G = _UPT // 128


def _topk_body(x_ref, idx_ref, val_ref):
    xcat = jnp.concatenate([x_ref[t, 0] for t in range(_T)], axis=1)
    nsq = jnp.sum(xcat * xcat, axis=1, keepdims=True)
    xn = xcat * jax.lax.rsqrt(nsq)
    acc = jax.lax.dot_general(
        xn, xn, (((1,), (1,)), ((), ())),
        preferred_element_type=jnp.float32)

    colf = jax.lax.broadcasted_iota(
        jnp.int32, (_N, _N), 1).astype(jnp.float32)
    rowf = jax.lax.broadcasted_iota(
        jnp.int32, (_N, 1), 0).astype(jnp.float32)
    work = acc
    jcols, vcols = [], []
    for _ in range(_K):
        m = jnp.max(work, axis=1, keepdims=True)
        eq = work >= m
        jf = jnp.max(jnp.where(eq, colf, -1.0), axis=1, keepdims=True)
        lv = jnp.where(m >= 0, m, 0.01 * m) * 0.5
        jcols.append(jf)
        vcols.append(lv)
        work = jnp.where(eq, -jnp.inf, work)
    is_self = jcols[0] == rowf
    dvals = [jnp.where(is_self, vcols[0] + vcols[0], vcols[0])] + vcols[1:]
    tvals = [jnp.where(is_self, 0.0, vcols[0])] + vcols[1:]
    fn = float(_N)
    icols = [rowf * fn + jf for jf in jcols]
    tcols = [jf * fn + rowf for jf in jcols]
    zf = jnp.zeros((_N, 1), jnp.float32)
    u_idx = jnp.concatenate(icols + tcols + [zf] * 6, axis=1)
    u_val = jnp.concatenate(dvals + tvals + [zf] * 6, axis=1)
    idx_ref[0] = u_idx.T.astype(jnp.int32)
    val_ref[0] = u_val.T


_NPLANES = 16


_RPT = _N // _NT
_NG = 4
_GB = _B // _NG


def _sc_scatter(idx_hbm, val_hbm, out_hbm, idx_v, val_v, chunk):
    c = jax.lax.axis_index("c")
    s = jax.lax.axis_index("s")
    base = s * _SLICE

    def zbody(i, carry):
        chunk[pl.ds(i * 16, 16)] = jnp.zeros((16,), jnp.float32)
        return carry

    jax.lax.fori_loop(0, _SLICE // 16, zbody, 0, unroll=8)

    for r in range(_GB // _NC):
        b = r * _NC + c
        pltpu.sync_copy(idx_hbm.at[b], idx_v)
        pltpu.sync_copy(val_hbm.at[b], val_v)

        for p in range(_K):
            def dbody(i, carry):
                j = (i + s * (_RPT // 16)) * 16
                iv = idx_v[p, pl.ds(j, 16)] - base
                vv = val_v[p, pl.ds(j, 16)]
                plsc.addupdate_scatter(chunk, [iv], vv)
                return carry
            jax.lax.fori_loop(0, _RPT // 16, dbody, 0, unroll=4)

            def tbody(i, carry):
                iv = idx_v[_K + p, pl.ds(i * 16, 16)] - base
                vv = val_v[_K + p, pl.ds(i * 16, 16)]
                msk = (iv >= 0) & (iv < _SLICE)
                loc = jnp.where(msk, iv, 0)
                plsc.addupdate_scatter(chunk, [loc], jnp.where(msk, vv, 0.0))
                return carry
            jax.lax.fori_loop(0, _N // 16, tbody, 0, unroll=8)

        pltpu.sync_copy(chunk, out_hbm.at[b, pl.ds(base, _SLICE)])
        jax.lax.fori_loop(0, _SLICE // 16, zbody, 0, unroll=8)


_scatter_call = pl.kernel(
    _sc_scatter,
    out_type=jax.ShapeDtypeStruct((_GB, _SLAB), jnp.float32),
    mesh=plsc.VectorSubcoreMesh(core_axis_name="c", subcore_axis_name="s"),
    compiler_params=pltpu.CompilerParams(needs_layout_passes=False),
    scratch_types=[
        pltpu.VMEM((_NPLANES, _N), jnp.int32),
        pltpu.VMEM((_NPLANES, _N), jnp.float32),
        pltpu.VMEM((_SLICE,), jnp.float32),
    ],
)


def _topk_call(x, g):
    return pl.pallas_call(
        _topk_body,
        grid=(_GB,),
        in_specs=[pl.BlockSpec((_T, 1, _N, _D),
                               lambda b, g=g: (0, g * _GB + b, 0, 0))],
        out_specs=[
            pl.BlockSpec((1, _NPLANES, _N), lambda b: (b, 0, 0)),
            pl.BlockSpec((1, _NPLANES, _N), lambda b: (b, 0, 0)),
        ],
        out_shape=[
            jax.ShapeDtypeStruct((_GB, _NPLANES, _N), jnp.int32),
            jax.ShapeDtypeStruct((_GB, _NPLANES, _N), jnp.float32),
        ],
    )(x)


def kernel(x):
    flats = []
    for g in range(_NG):
        idx, val = _topk_call(x, g)
        flats.append(_scatter_call(idx, val))
    return jnp.concatenate(flats, axis=0).reshape(_B, _N, _N)
